# baseline (speedup 1.0000x reference)
import functools

import jax
import jax.numpy as jnp
from jax import lax
from jax.experimental import pallas as pl
from jax.experimental.pallas import tpu as pltpu
from jax.experimental.pallas import tpu_sc as plsc

N = 10000
E = 320000
D = 128

NC = 2
NS = 16
NW = NC * NS
EW = E // NW
C = 25
NBUF = 4
CBLK = 40
NBLK = EW // (C * CBLK)
NGRP = CBLK // NBUF
STRIPE = 640
NV = D // 16


def _sc_kernel(src4, dst4, w4, h):
    mesh = plsc.VectorSubcoreMesh(core_axis_name="c", subcore_axis_name="s")

    @functools.partial(
        pl.kernel,
        out_type=jax.ShapeDtypeStruct((NC, N, D), jnp.float32),
        mesh=mesh,
        scratch_types=[
            pltpu.VMEM((CBLK, C), jnp.int32),
            pltpu.VMEM((CBLK, C), jnp.int32),
            pltpu.VMEM((CBLK, C), jnp.float32),
            [pltpu.VMEM((C, D), jnp.float32)] * NBUF,
            [pltpu.VMEM((C, D), jnp.float32)] * NBUF,
            pltpu.VMEM_SHARED((N, D), jnp.float32),
            [pltpu.SemaphoreType.DMA] * NBUF,
            [pltpu.SemaphoreType.DMA] * NBUF,
        ],
        compiler_params=pltpu.CompilerParams(needs_layout_passes=False),
    )
    def k(src_hbm, dst_hbm, w_hbm, h_hbm, out_hbm,
          srcv, dstv, wv, grows, srows, acc, gsem, ssem):
        cid = lax.axis_index("c")
        sid = lax.axis_index("s")
        wid = sid * NC + cid

        def _zrow(j, _):
            for k2 in range(NV):
                grows[0][j, pl.ds(k2 * 16, 16)] = jnp.zeros((16,), jnp.float32)
            return ()
        lax.fori_loop(0, C, _zrow, ())
        base = sid * STRIPE
        nrows = jnp.where(sid == NS - 1, N - (NS - 1) * STRIPE, STRIPE)

        def _zcp(r, _):
            pltpu.sync_copy(grows[0].at[pl.ds(0, 16)],
                            acc.at[pl.ds(base + r * 16, 16)])
            return ()
        lax.fori_loop(0, nrows // 16, _zcp, ())
        plsc.subcore_barrier()

        def _scale(c, b):
            cvec = jnp.full((16,), c, jnp.int32)

            def _edge2(j2, __):
                j = j2 * 2
                wb0 = plsc.load_gather(
                    wv, [cvec, jnp.full((16,), j, jnp.int32)])
                wb1 = plsc.load_gather(
                    wv, [cvec, jnp.full((16,), j + 1, jnp.int32)])
                for k2 in range(NV):
                    sl = pl.ds(k2 * 16, 16)
                    srows[b][j, sl] = grows[b][j, sl] * wb0
                for k2 in range(NV):
                    sl = pl.ds(k2 * 16, 16)
                    srows[b][j + 1, sl] = grows[b][j + 1, sl] * wb1
                return ()
            lax.fori_loop(0, C // 2, _edge2, ())

        def _start_gather(c, b):
            pltpu.async_copy(h_hbm.at[srcv.at[c]], grows[b], gsem[b])

        def _wait_gather(b):
            pltpu.make_async_copy(h_hbm.at[srcv.at[0]], grows[b],
                                  gsem[b]).wait()

        def _start_scatter(c, b):
            pltpu.async_copy(srows[b], acc.at[dstv.at[c]], ssem[b], add=True)

        def _wait_scatter(b):
            pltpu.make_async_copy(srows[b], acc.at[dstv.at[0]],
                                  ssem[b]).wait()

        def _block(r, _):
            pltpu.sync_copy(src_hbm.at[wid, r], srcv)
            pltpu.sync_copy(dst_hbm.at[wid, r], dstv)
            pltpu.sync_copy(w_hbm.at[wid, r], wv)

            for b in range(NBUF):
                _start_gather(b, b)

            for b in range(NBUF):
                _wait_gather(b)
                _scale(b, b)
                _start_scatter(b, b)
                _start_gather(NBUF + b, b)

            def _group(q, _):
                for b in range(NBUF):
                    c = q * NBUF + b
                    _wait_gather(b)
                    _wait_scatter(b)
                    _scale(c, b)
                    _start_scatter(c, b)
                    _start_gather(c + NBUF, b)
                return ()
            lax.fori_loop(1, NGRP - 1, _group, ())

            for b in range(NBUF):
                c = (NGRP - 1) * NBUF + b
                _wait_gather(b)
                _wait_scatter(b)
                _scale(c, b)
                _start_scatter(c, b)
            for b in range(NBUF):
                _wait_scatter(b)
            return ()
        lax.fori_loop(0, NBLK, _block, ())

        plsc.subcore_barrier()

        def _ocp(r, _):
            pltpu.sync_copy(acc.at[pl.ds(base + r * 80, 80)],
                            out_hbm.at[cid, pl.ds(base + r * 80, 80)])
            return ()
        lax.fori_loop(0, nrows // 80, _ocp, ())

    return k(src4, dst4, w4, h)


def _mm_body(x_ref, w_ref, o_ref):
    o_ref[...] = jnp.dot(x_ref[...], w_ref[...],
                         preferred_element_type=jnp.float32)


def _combine_body(p_ref, b_ref, o_ref):
    o_ref[...] = p_ref[0] + p_ref[1] + b_ref[...]


@jax.jit
def kernel(x, edge_index, edge_weight, W, b):
    MB = 1000
    h = pl.pallas_call(
        _mm_body,
        grid=(N // MB,),
        in_specs=[
            pl.BlockSpec((MB, D), lambda i: (i, 0)),
            pl.BlockSpec((D, D), lambda i: (0, 0)),
        ],
        out_specs=pl.BlockSpec((MB, D), lambda i: (i, 0)),
        out_shape=jax.ShapeDtypeStruct((N, D), jnp.float32),
    )(x, W)

    dst4 = edge_index[0].reshape(NW, NBLK, CBLK, C)
    src4 = edge_index[1].reshape(NW, NBLK, CBLK, C)
    w4 = edge_weight.reshape(NW, NBLK, CBLK, C)

    partials = _sc_kernel(src4, dst4, w4, h)

    out = pl.pallas_call(
        _combine_body,
        grid=(N // MB,),
        in_specs=[
            pl.BlockSpec((NC, MB, D), lambda i: (0, i, 0)),
            pl.BlockSpec((1, D), lambda i: (0, 0)),
        ],
        out_specs=pl.BlockSpec((MB, D), lambda i: (i, 0)),
        out_shape=jax.ShapeDtypeStruct((N, D), jnp.float32),
    )(partials, b.reshape(1, D))
    return out


# speedup vs baseline: 1.7707x; 1.7707x over previous
import functools

import jax
import jax.numpy as jnp
from jax import lax
from jax.experimental import pallas as pl
from jax.experimental.pallas import tpu as pltpu
from jax.experimental.pallas import tpu_sc as plsc

N = 10000
E = 320000
D = 128

NC = 2
NS = 16
NW = NC * NS
EW = E // NW
C = 50
NBUF = 4
CBLK = 40
NBLK = EW // (C * CBLK)
NGRP = CBLK // NBUF
STRIPE = 640
NV = D // 16


def _sc_kernel(src4, dst4, w4, h):
    mesh = plsc.VectorSubcoreMesh(core_axis_name="c", subcore_axis_name="s")

    @functools.partial(
        pl.kernel,
        out_type=jax.ShapeDtypeStruct((NC, N, D), jnp.float32),
        mesh=mesh,
        scratch_types=[
            pltpu.VMEM((CBLK, C), jnp.int32),
            pltpu.VMEM((CBLK, C), jnp.int32),
            pltpu.VMEM((CBLK, C), jnp.float32),
            [pltpu.VMEM((C, D), jnp.float32)] * NBUF,
            pltpu.VMEM_SHARED((N, D), jnp.float32),
            [pltpu.SemaphoreType.DMA] * NBUF,
            [pltpu.SemaphoreType.DMA] * NBUF,
        ],
        compiler_params=pltpu.CompilerParams(needs_layout_passes=False),
    )
    def k(src_hbm, dst_hbm, w_hbm, h_hbm, out_hbm,
          srcv, dstv, wv, rows, acc, gsem, ssem):
        cid = lax.axis_index("c")
        sid = lax.axis_index("s")
        wid = sid * NC + cid

        def _zrow(j, _):
            for k2 in range(NV):
                rows[0][j, pl.ds(k2 * 16, 16)] = jnp.zeros((16,), jnp.float32)
            return ()
        lax.fori_loop(0, C, _zrow, ())
        base = sid * STRIPE
        nrows = jnp.where(sid == NS - 1, N - (NS - 1) * STRIPE, STRIPE)

        def _zcp(r, _):
            pltpu.sync_copy(rows[0].at[pl.ds(0, 16)],
                            acc.at[pl.ds(base + r * 16, 16)])
            return ()
        lax.fori_loop(0, nrows // 16, _zcp, ())
        plsc.subcore_barrier()

        def _scale(c, b):
            cvec = jnp.full((16,), c, jnp.int32)

            def _edge2(j2, __):
                j = j2 * 2
                wb0 = plsc.load_gather(
                    wv, [cvec, jnp.full((16,), j, jnp.int32)])
                wb1 = plsc.load_gather(
                    wv, [cvec, jnp.full((16,), j + 1, jnp.int32)])
                for k2 in range(NV):
                    sl = pl.ds(k2 * 16, 16)
                    rows[b][j, sl] = rows[b][j, sl] * wb0
                for k2 in range(NV):
                    sl = pl.ds(k2 * 16, 16)
                    rows[b][j + 1, sl] = rows[b][j + 1, sl] * wb1
                return ()
            lax.fori_loop(0, C // 2, _edge2, ())

        def _start_gather(c, b):
            pltpu.async_copy(h_hbm.at[srcv.at[c]], rows[b], gsem[b])

        def _wait_gather(b):
            pltpu.make_async_copy(h_hbm.at[srcv.at[0]], rows[b], gsem[b]).wait()

        def _start_scatter(c, b):
            pltpu.async_copy(rows[b], acc.at[dstv.at[c]], ssem[b], add=True)

        def _wait_scatter(b):
            pltpu.make_async_copy(rows[b], acc.at[dstv.at[0]], ssem[b]).wait()

        for r in range(NBLK):
            pltpu.sync_copy(src_hbm.at[wid, r], srcv)
            pltpu.sync_copy(dst_hbm.at[wid, r], dstv)
            pltpu.sync_copy(w_hbm.at[wid, r], wv)

            for i in range(2):
                _start_gather(i, i)
            for i in range(2):
                _wait_gather(i)
                _scale(i, i)
                _start_scatter(i, i)
            for i in range(2):
                _start_gather(2 + i, 2 + i)
            for i in range(2):
                _wait_gather(2 + i)
                _scale(2 + i, 2 + i)
                _start_scatter(2 + i, 2 + i)
            for i in range(2):
                _wait_scatter(i)
                _start_gather(4 + i, i)

            def _super(g, _):
                c0 = g * 4
                for i in range(2):
                    _wait_gather(i)
                    _scale(c0 + i, i)
                    _start_scatter(c0 + i, i)
                for i in range(2):
                    _wait_scatter(2 + i)
                    _start_gather(c0 + 2 + i, 2 + i)
                for i in range(2):
                    _wait_gather(2 + i)
                    _scale(c0 + 2 + i, 2 + i)
                    _start_scatter(c0 + 2 + i, 2 + i)
                for i in range(2):
                    _wait_scatter(i)
                    _start_gather(c0 + 4 + i, i)
                return ()
            lax.fori_loop(1, NGRP - 1, _super, ())

            cl = (NGRP - 1) * 4
            for i in range(2):
                _wait_gather(i)
                _scale(cl + i, i)
                _start_scatter(cl + i, i)
            for i in range(2):
                _wait_scatter(2 + i)
                _start_gather(cl + 2 + i, 2 + i)
            for i in range(2):
                _wait_gather(2 + i)
                _scale(cl + 2 + i, 2 + i)
                _start_scatter(cl + 2 + i, 2 + i)
            for i in range(2):
                _wait_scatter(i)
            for i in range(2):
                _wait_scatter(2 + i)

        plsc.subcore_barrier()

        def _ocp(r, _):
            pltpu.sync_copy(acc.at[pl.ds(base + r * 80, 80)],
                            out_hbm.at[cid, pl.ds(base + r * 80, 80)])
            return ()
        lax.fori_loop(0, nrows // 80, _ocp, ())

    return k(src4, dst4, w4, h)


def _mm_body(x_ref, w_ref, o_ref):
    o_ref[...] = jnp.dot(x_ref[...], w_ref[...],
                         preferred_element_type=jnp.float32)


def _combine_body(p_ref, b_ref, o_ref):
    o_ref[...] = p_ref[0] + p_ref[1] + b_ref[...]


@jax.jit
def kernel(x, edge_index, edge_weight, W, b):
    MB = 1000
    h = pl.pallas_call(
        _mm_body,
        grid=(N // MB,),
        in_specs=[
            pl.BlockSpec((MB, D), lambda i: (i, 0)),
            pl.BlockSpec((D, D), lambda i: (0, 0)),
        ],
        out_specs=pl.BlockSpec((MB, D), lambda i: (i, 0)),
        out_shape=jax.ShapeDtypeStruct((N, D), jnp.float32),
    )(x, W)

    dst4 = edge_index[0].reshape(NW, NBLK, CBLK, C)
    src4 = edge_index[1].reshape(NW, NBLK, CBLK, C)
    w4 = edge_weight.reshape(NW, NBLK, CBLK, C)

    partials = _sc_kernel(src4, dst4, w4, h)

    out = pl.pallas_call(
        _combine_body,
        grid=(N // MB,),
        in_specs=[
            pl.BlockSpec((NC, MB, D), lambda i: (0, i, 0)),
            pl.BlockSpec((1, D), lambda i: (0, 0)),
        ],
        out_specs=pl.BlockSpec((MB, D), lambda i: (i, 0)),
        out_shape=jax.ShapeDtypeStruct((N, D), jnp.float32),
    )(partials, b.reshape(1, D))
    return out


# speedup vs baseline: 1.9720x; 1.1137x over previous
import functools

import jax
import jax.numpy as jnp
from jax import lax
from jax.experimental import pallas as pl
from jax.experimental.pallas import tpu as pltpu
from jax.experimental.pallas import tpu_sc as plsc

N = 10000
E = 320000
D = 128

NC = 2
NS = 16
NW = NC * NS
EW = E // NW
C = 50
NBUF = 4
CBLK = 40
NBLK = EW // (C * CBLK)
NGRP = CBLK // NBUF
STRIPE = 640
NV = D // 16


def _sc_kernel(src4, dst4, w4, h):
    mesh = plsc.VectorSubcoreMesh(core_axis_name="c", subcore_axis_name="s")

    @functools.partial(
        pl.kernel,
        out_type=jax.ShapeDtypeStruct((NC, N, D), jnp.float32),
        mesh=mesh,
        scratch_types=[
            pltpu.VMEM((CBLK, C), jnp.int32),
            pltpu.VMEM((CBLK, C), jnp.int32),
            pltpu.VMEM((CBLK, C), jnp.float32),
            [pltpu.VMEM((C, D), jnp.float32)] * NBUF,
            pltpu.VMEM_SHARED((N, D), jnp.float32),
            [pltpu.SemaphoreType.DMA] * NBUF,
            [pltpu.SemaphoreType.DMA] * NBUF,
        ],
        compiler_params=pltpu.CompilerParams(needs_layout_passes=False),
    )
    def k(src_hbm, dst_hbm, w_hbm, h_hbm, out_hbm,
          srcv, dstv, wv, rows, acc, gsem, ssem):
        cid = lax.axis_index("c")
        sid = lax.axis_index("s")
        wid = sid * NC + cid

        def _zrow(j, _):
            for k2 in range(NV):
                rows[0][j, pl.ds(k2 * 16, 16)] = jnp.zeros((16,), jnp.float32)
            return ()
        lax.fori_loop(0, C, _zrow, ())
        base = sid * STRIPE
        nrows = jnp.where(sid == NS - 1, N - (NS - 1) * STRIPE, STRIPE)

        def _zcp(r, _):
            pltpu.sync_copy(rows[0].at[pl.ds(0, 16)],
                            acc.at[pl.ds(base + r * 16, 16)])
            return ()
        lax.fori_loop(0, nrows // 16, _zcp, ())
        plsc.subcore_barrier()

        def _scale(c, b):
            cvec = jnp.full((16,), c, jnp.int32)

            def _edge2(j2, __):
                j = j2 * 2
                wb0 = plsc.load_gather(
                    wv, [cvec, jnp.full((16,), j, jnp.int32)])
                wb1 = plsc.load_gather(
                    wv, [cvec, jnp.full((16,), j + 1, jnp.int32)])
                for k2 in range(NV):
                    sl = pl.ds(k2 * 16, 16)
                    rows[b][j, sl] = rows[b][j, sl] * wb0
                for k2 in range(NV):
                    sl = pl.ds(k2 * 16, 16)
                    rows[b][j + 1, sl] = rows[b][j + 1, sl] * wb1
                return ()
            lax.fori_loop(0, C // 2, _edge2, ())

        def _start_gather(c, b):
            pltpu.async_copy(h_hbm.at[srcv.at[c]], rows[b], gsem[b])

        def _wait_gather(b):
            pltpu.make_async_copy(h_hbm.at[srcv.at[0]], rows[b], gsem[b]).wait()

        def _start_scatter(c, b):
            pltpu.async_copy(rows[b], acc.at[dstv.at[c]], ssem[b], add=True)

        def _wait_scatter(b):
            pltpu.make_async_copy(rows[b], acc.at[dstv.at[0]], ssem[b]).wait()

        for r in range(NBLK):
            pltpu.sync_copy(src_hbm.at[wid, r], srcv)
            pltpu.sync_copy(dst_hbm.at[wid, r], dstv)
            pltpu.sync_copy(w_hbm.at[wid, r], wv)

            for b in range(NBUF):
                _start_gather(b, b)

            def _group(q, _):
                for b in range(NBUF):
                    c = q * NBUF + b
                    _wait_gather(b)
                    _scale(c, b)
                    _start_scatter(c, b)
                for b in range(NBUF):
                    _wait_scatter(b)
                    _start_gather((q + 1) * NBUF + b, b)
                return ()
            lax.fori_loop(0, NGRP - 1, _group, ())

            for b in range(NBUF):
                c = (NGRP - 1) * NBUF + b
                _wait_gather(b)
                _scale(c, b)
                _start_scatter(c, b)
            for b in range(NBUF):
                _wait_scatter(b)

        plsc.subcore_barrier()

        def _ocp(r, _):
            pltpu.sync_copy(acc.at[pl.ds(base + r * 80, 80)],
                            out_hbm.at[cid, pl.ds(base + r * 80, 80)])
            return ()
        lax.fori_loop(0, nrows // 80, _ocp, ())

    return k(src4, dst4, w4, h)


def _mm_body(x_ref, w_ref, o_ref):
    o_ref[...] = jnp.dot(x_ref[...], w_ref[...],
                         preferred_element_type=jnp.float32)


def _combine_body(p_ref, b_ref, o_ref):
    o_ref[...] = p_ref[0] + p_ref[1] + b_ref[...]


@jax.jit
def kernel(x, edge_index, edge_weight, W, b):
    MB = 1000
    h = pl.pallas_call(
        _mm_body,
        grid=(N // MB,),
        in_specs=[
            pl.BlockSpec((MB, D), lambda i: (i, 0)),
            pl.BlockSpec((D, D), lambda i: (0, 0)),
        ],
        out_specs=pl.BlockSpec((MB, D), lambda i: (i, 0)),
        out_shape=jax.ShapeDtypeStruct((N, D), jnp.float32),
    )(x, W)

    dst4 = edge_index[0].reshape(NW, NBLK, CBLK, C)
    src4 = edge_index[1].reshape(NW, NBLK, CBLK, C)
    w4 = edge_weight.reshape(NW, NBLK, CBLK, C)

    partials = _sc_kernel(src4, dst4, w4, h)

    out = pl.pallas_call(
        _combine_body,
        grid=(N // MB,),
        in_specs=[
            pl.BlockSpec((NC, MB, D), lambda i: (0, i, 0)),
            pl.BlockSpec((1, D), lambda i: (0, 0)),
        ],
        out_specs=pl.BlockSpec((MB, D), lambda i: (i, 0)),
        out_shape=jax.ShapeDtypeStruct((N, D), jnp.float32),
    )(partials, b.reshape(1, D))
    return out


# speedup vs baseline: 2.1021x; 1.0660x over previous
import functools

import jax
import jax.numpy as jnp
from jax import lax
from jax.experimental import pallas as pl
from jax.experimental.pallas import tpu as pltpu
from jax.experimental.pallas import tpu_sc as plsc

N = 10000
E = 320000
D = 128

NC = 2
NS = 16
NW = NC * NS
EW = E // NW
C = 50
NBUF = 4
CBLK = 40
NBLK = EW // (C * CBLK)
NGRP = CBLK // NBUF
STRIPE = 640
NV = D // 16


def _sc_kernel(src4, dst4, w4, h):
    mesh = plsc.VectorSubcoreMesh(core_axis_name="c", subcore_axis_name="s")

    @functools.partial(
        pl.kernel,
        out_type=jax.ShapeDtypeStruct((NC, N, D), jnp.float32),
        mesh=mesh,
        scratch_types=[
            pltpu.VMEM((CBLK, C), jnp.int32),
            pltpu.VMEM((CBLK, C), jnp.int32),
            pltpu.VMEM((CBLK, C), jnp.float32),
            [pltpu.VMEM((C, D), jnp.float32)] * NBUF,
            pltpu.VMEM_SHARED((N, D), jnp.float32),
            [pltpu.SemaphoreType.DMA] * NBUF,
            [pltpu.SemaphoreType.DMA] * NBUF,
        ],
        compiler_params=pltpu.CompilerParams(needs_layout_passes=False),
    )
    def k(src_hbm, dst_hbm, w_hbm, h_hbm, out_hbm,
          srcv, dstv, wv, rows, acc, gsem, ssem):
        cid = lax.axis_index("c")
        sid = lax.axis_index("s")
        wid = sid * NC + cid

        def _zrow(j, _):
            for k2 in range(NV):
                rows[0][j, pl.ds(k2 * 16, 16)] = jnp.zeros((16,), jnp.float32)
            return ()
        lax.fori_loop(0, C, _zrow, ())
        base = sid * STRIPE
        nrows = jnp.where(sid == NS - 1, N - (NS - 1) * STRIPE, STRIPE)

        def _zcp(r, _):
            pltpu.sync_copy(rows[0].at[pl.ds(0, 16)],
                            acc.at[pl.ds(base + r * 16, 16)])
            return ()
        lax.fori_loop(0, nrows // 16, _zcp, ())
        plsc.subcore_barrier()

        def _scale(c, b):
            cvec = jnp.full((16,), c, jnp.int32)

            UNR = 5
            def _edgeu(j2, __):
                j = j2 * UNR
                wbs = [plsc.load_gather(
                    wv, [cvec, jnp.full((16,), j + u, jnp.int32)])
                    for u in range(UNR)]
                for u in range(UNR):
                    for k2 in range(NV):
                        sl = pl.ds(k2 * 16, 16)
                        rows[b][j + u, sl] = rows[b][j + u, sl] * wbs[u]
                return ()
            lax.fori_loop(0, C // UNR, _edgeu, ())

        def _start_gather(c, b):
            pltpu.async_copy(h_hbm.at[srcv.at[c]], rows[b], gsem[b])

        def _wait_gather(b):
            pltpu.make_async_copy(h_hbm.at[srcv.at[0]], rows[b], gsem[b]).wait()

        def _start_scatter(c, b):
            pltpu.async_copy(rows[b], acc.at[dstv.at[c]], ssem[b], add=True)

        def _wait_scatter(b):
            pltpu.make_async_copy(rows[b], acc.at[dstv.at[0]], ssem[b]).wait()

        for r in range(NBLK):
            pltpu.sync_copy(src_hbm.at[wid, r], srcv)
            pltpu.sync_copy(dst_hbm.at[wid, r], dstv)
            pltpu.sync_copy(w_hbm.at[wid, r], wv)

            for b in range(NBUF):
                _start_gather(b, b)

            def _group(q, _):
                for b in range(NBUF):
                    c = q * NBUF + b
                    _wait_gather(b)
                    _scale(c, b)
                    _start_scatter(c, b)
                for b in range(NBUF):
                    _wait_scatter(b)
                    _start_gather((q + 1) * NBUF + b, b)
                return ()
            lax.fori_loop(0, NGRP - 1, _group, ())

            for b in range(NBUF):
                c = (NGRP - 1) * NBUF + b
                _wait_gather(b)
                _scale(c, b)
                _start_scatter(c, b)
            for b in range(NBUF):
                _wait_scatter(b)

        plsc.subcore_barrier()

        def _ocp(r, _):
            pltpu.sync_copy(acc.at[pl.ds(base + r * 80, 80)],
                            out_hbm.at[cid, pl.ds(base + r * 80, 80)])
            return ()
        lax.fori_loop(0, nrows // 80, _ocp, ())

    return k(src4, dst4, w4, h)


def _mm_body(x_ref, w_ref, o_ref):
    o_ref[...] = jnp.dot(x_ref[...], w_ref[...],
                         preferred_element_type=jnp.float32)


def _combine_body(p_ref, b_ref, o_ref):
    o_ref[...] = p_ref[0] + p_ref[1] + b_ref[...]


@jax.jit
def kernel(x, edge_index, edge_weight, W, b):
    MB = 1000
    h = pl.pallas_call(
        _mm_body,
        grid=(N // MB,),
        in_specs=[
            pl.BlockSpec((MB, D), lambda i: (i, 0)),
            pl.BlockSpec((D, D), lambda i: (0, 0)),
        ],
        out_specs=pl.BlockSpec((MB, D), lambda i: (i, 0)),
        out_shape=jax.ShapeDtypeStruct((N, D), jnp.float32),
    )(x, W)

    dst4 = edge_index[0].reshape(NW, NBLK, CBLK, C)
    src4 = edge_index[1].reshape(NW, NBLK, CBLK, C)
    w4 = edge_weight.reshape(NW, NBLK, CBLK, C)

    partials = _sc_kernel(src4, dst4, w4, h)

    out = pl.pallas_call(
        _combine_body,
        grid=(N // MB,),
        in_specs=[
            pl.BlockSpec((NC, MB, D), lambda i: (0, i, 0)),
            pl.BlockSpec((1, D), lambda i: (0, 0)),
        ],
        out_specs=pl.BlockSpec((MB, D), lambda i: (i, 0)),
        out_shape=jax.ShapeDtypeStruct((N, D), jnp.float32),
    )(partials, b.reshape(1, D))
    return out
